# baseline (speedup 1.0000x reference)
import functools

import jax
import jax.numpy as jnp
from jax import lax
from jax.experimental import pallas as pl
from jax.experimental.pallas import tpu as pltpu
from jax.experimental.pallas import tpu_sc as plsc

N = 10000
E = 320000
D_IN = 128
H = 64
C = 100
B = 16

NC = 2
NS = 16
NP = 10240
NPT = NP // NS
EN = E + N
CHUNK = 128
TCH = 81
NW = NC * NS
NROWS = NW * TCH
EP = NROWS * CHUNK
T0 = 94
T1 = 68
TMAX = 94

_SC_PARAMS = pltpu.CompilerParams(needs_layout_passes=False,
                                 use_tc_tiling_on_sc=False)

_mesh = functools.partial(
    plsc.VectorSubcoreMesh, core_axis_name="c", subcore_axis_name="s")

_f32 = jnp.float32
_i32 = jnp.int32


def _wid_base(scale):
  c = lax.axis_index("c")
  s = lax.axis_index("s")
  return (c * NS + s) * scale, c, s


def _deg_body(colr, degp, col_v, val_v, zero_v, acc):
  base, c, s = _wid_base(TCH)
  w = c * NS + s

  def zstore(i, _):
    zero_v[pl.ds(i * 16, 16)] = jnp.zeros((16,), _f32)
    return 0
  lax.fori_loop(0, NPT // 16, zstore, 0)
  pltpu.sync_copy(zero_v, acc.at[pl.ds(s * NPT, NPT)])
  plsc.subcore_barrier()

  pltpu.sync_copy(colr.at[pl.ds(w * TCH, TCH)], col_v)

  def chunk(j, _):
    gbase = (base + j) * CHUNK

    def fill(k, _):
      idx = gbase + k * 16 + lax.iota(_i32, 16)
      val_v[pl.ds(k * 16, 16)] = jnp.where(idx < EN, jnp.ones((16,), _f32),
                                           jnp.zeros((16,), _f32))
      return 0
    lax.fori_loop(0, CHUNK // 16, fill, 0)
    pltpu.sync_copy(val_v, acc.at[col_v.at[j, 0]], add=True)
    return 0
  lax.fori_loop(0, TCH, chunk, 0)
  plsc.subcore_barrier()
  pltpu.sync_copy(acc.at[pl.ds(s * NPT, NPT)],
                  degp.at[c, pl.ds(s * NPT, NPT)])


_deg_call = pl.kernel(
    _deg_body,
    out_type=jax.ShapeDtypeStruct((NC, NP), _f32),
    mesh=_mesh(),
    compiler_params=_SC_PARAMS,
    scratch_types=[
        pltpu.VMEM((TCH, 1, CHUNK), _i32),
        pltpu.VMEM((CHUNK,), _f32),
        pltpu.VMEM((NPT,), _f32),
        pltpu.VMEM_SHARED((NP,), _f32),
    ],
)


def _norm_body(rowr, colr, ear, dis_hbm, normr,
               row_v, col_v, ea_v, dis_v, nbuf):
  base, c, s = _wid_base(TCH)
  w = c * NS + s
  pltpu.sync_copy(dis_hbm, dis_v)
  pltpu.sync_copy(rowr.at[pl.ds(w * TCH, TCH)], row_v)
  pltpu.sync_copy(colr.at[pl.ds(w * TCH, TCH)], col_v)
  pltpu.sync_copy(ear.at[pl.ds(w * TCH, TCH)], ea_v)

  def chunk(j, _):
    def sub(k, _):
      r16 = row_v[j, 0, pl.ds(k * 16, 16)]
      c16 = col_v[j, 0, pl.ds(k * 16, 16)]
      e16 = ea_v[j, 0, pl.ds(k * 16, 16)]
      dr = plsc.load_gather(dis_v, [r16])
      dc = plsc.load_gather(dis_v, [c16])
      g = (base + j) * CHUNK + k * 16 + lax.iota(_i32, 16)
      nr = dr * dc * jnp.exp(-e16)
      nbuf[j, 0, pl.ds(k * 16, 16)] = jnp.where(g < EN, nr,
                                                jnp.zeros((16,), _f32))
      return 0
    lax.fori_loop(0, CHUNK // 16, sub, 0)
    return 0
  lax.fori_loop(0, TCH, chunk, 0)
  pltpu.sync_copy(nbuf, normr.at[pl.ds(w * TCH, TCH)])


_norm_call = pl.kernel(
    _norm_body,
    out_type=jax.ShapeDtypeStruct((NROWS, 1, CHUNK), _f32),
    mesh=_mesh(),
    compiler_params=_SC_PARAMS,
    scratch_types=[
        pltpu.VMEM((TCH, 1, CHUNK), _i32),
        pltpu.VMEM((TCH, 1, CHUNK), _i32),
        pltpu.VMEM((TCH, 1, CHUNK), _f32),
        pltpu.VMEM((NP,), _f32),
        pltpu.VMEM((TCH, 1, CHUNK), _f32),
    ],
)


NGB = 3
NSB = 3
PD = 2


def _spmm_body(hw, rowr, colr, normr, s_out,
               row_v, col_v, norm_v, gb, sb, zb, acc, gsem, ssem):
  base, c, s = _wid_base(TCH)
  myt = jnp.where(c == 0, T0, T1)
  start = jnp.where(c == 0, s * T0, 16 * T0 + s * T1)

  def zfill(i, _):
    for f in range(4):
      zb[i, pl.ds(f * 16, 16)] = jnp.zeros((16,), _f32)
    return 0
  lax.fori_loop(0, 16, zfill, 0)

  def zfire(t, _):
    pltpu.async_copy(zb, acc.at[pl.ds(s * NPT + t * 16, 16), :], gsem)
    return 0
  lax.fori_loop(0, NPT // 16, zfire, 0)

  def zdrain(t, _):
    pltpu.make_async_copy(zb, acc.at[pl.ds(s * NPT + t * 16, 16), :],
                          gsem).wait()
    return 0
  lax.fori_loop(0, NPT // 16, zdrain, 0)

  pltpu.sync_copy(rowr.at[pl.ds(start, TMAX)], row_v)
  pltpu.sync_copy(colr.at[pl.ds(start, TMAX)], col_v)
  pltpu.sync_copy(normr.at[pl.ds(start, TMAX)], norm_v)
  plsc.subcore_barrier()

  def fire_g(j, bf):
    pltpu.async_copy(hw.at[row_v.at[j, 0]], gb.at[bf], gsem)

  def drain_g(j, bf):
    pltpu.make_async_copy(hw.at[row_v.at[j, 0]], gb.at[bf], gsem).wait()

  def fire_s(j, bf):
    pltpu.async_copy(sb.at[bf], acc.at[col_v.at[j, 0]], ssem, add=True)

  def drain_s(j, bf):
    pltpu.make_async_copy(sb.at[bf], acc.at[col_v.at[j, 0]], ssem).wait()

  for j in range(PD):
    fire_g(j, j)

  def body(j, _):
    gf = j % NGB
    sf = j % NSB

    @pl.when(j >= NSB)
    def _():
      drain_s(j - NSB, sf)

    @pl.when(j + PD < myt)
    def _():
      fire_g(j + PD, (j + PD) % NGB)
    drain_g(j, gf)

    @plsc.parallel_loop(0, CHUNK // 16, unroll=2)
    def _(k):
      nv = norm_v[j, 0, pl.ds(k * 16, 16)]
      for l in range(16):
        nb = nv[jnp.full((16,), l, _i32)]
        r = k * 16 + l
        for f in range(4):
          sb[sf, r, pl.ds(f * 16, 16)] = nb * gb[gf, r, pl.ds(f * 16, 16)]
    fire_s(j, sf)
    return 0
  lax.fori_loop(0, myt, body, 0)
  for jj in range(NSB):
    j = myt - NSB + jj
    drain_s(j, j % NSB)

  plsc.subcore_barrier()
  pltpu.sync_copy(acc.at[pl.ds(s * NPT, NPT), :],
                  s_out.at[c, pl.ds(s * NPT, NPT), :])


_spmm_call = pl.kernel(
    _spmm_body,
    out_type=jax.ShapeDtypeStruct((NC, NP, H), _f32),
    mesh=_mesh(),
    compiler_params=_SC_PARAMS,
    scratch_types=[
        pltpu.VMEM((TMAX, 1, CHUNK), _i32),
        pltpu.VMEM((TMAX, 1, CHUNK), _i32),
        pltpu.VMEM((TMAX, 1, CHUNK), _f32),
        pltpu.VMEM((NGB, CHUNK, H), _f32),
        pltpu.VMEM((NSB, CHUNK, H), _f32),
        pltpu.VMEM((16, H), _f32),
        pltpu.VMEM_SHARED((NP, H), _f32),
        pltpu.SemaphoreType.DMA,
        pltpu.SemaphoreType.DMA,
    ],
)


_DN = (((1,), (1,)), ((), ()))


def _tc1_body(x_ref, w1_ref, degp_ref, hw_ref, dis_ref):
  deg = degp_ref[0:1, :] + degp_ref[1:2, :]
  dis_ref[...] = lax.rsqrt(deg)
  hw_ref[...] = lax.dot_general(x_ref[...], w1_ref[...], _DN,
                                preferred_element_type=_f32)


def _tc_layer_body(s_ref, b_ref, w_ref, out_ref):
  h = s_ref[0, :N, :] + s_ref[1, :N, :] + b_ref[...]
  h = jnp.maximum(h, 0.0)
  out_ref[...] = lax.dot_general(h, w_ref[...], _DN,
                                 preferred_element_type=_f32)


def _tc_final_body(s_ref, b_ref, batch_ref, wc_ref, bc_ref, out_ref):
  h = s_ref[0, :N, :] + s_ref[1, :N, :] + b_ref[...]
  h = jnp.maximum(h, 0.0)
  bt = batch_ref[...]
  oht = (bt == lax.broadcasted_iota(_i32, (B, N), 0)).astype(_f32)
  sums = lax.dot_general(oht, h, (((1,), (0,)), ((), ())),
                         preferred_element_type=_f32)
  cnt = jnp.sum(oht, axis=1, keepdims=True)
  pooled = sums / jnp.maximum(cnt, 1.0)
  out_ref[...] = lax.dot_general(pooled, wc_ref[...], _DN,
                                 preferred_element_type=_f32) + bc_ref[...]


_tc1_call = pl.pallas_call(
    _tc1_body,
    out_shape=(jax.ShapeDtypeStruct((N, H), _f32),
               jax.ShapeDtypeStruct((1, NP), _f32)),
)

_tc_layer_call = pl.pallas_call(
    _tc_layer_body,
    out_shape=jax.ShapeDtypeStruct((N, H), _f32),
)

_tc_final_call = pl.pallas_call(
    _tc_final_body,
    out_shape=jax.ShapeDtypeStruct((B, C), _f32),
)


def kernel(x, edge_index, edge_attr, batch, W1, b1, W2, b2, W3, b3, Wc, bc):
  loop = jnp.arange(N, dtype=_i32)
  zpad_i = jnp.zeros((EP - EN,), _i32)
  row = jnp.concatenate([edge_index[0], loop, zpad_i]).reshape(NROWS, 1, CHUNK)
  col = jnp.concatenate([edge_index[1], loop, zpad_i]).reshape(NROWS, 1, CHUNK)
  ea = jnp.concatenate([edge_attr,
                        jnp.zeros((EP - E,), _f32)]).reshape(NROWS, 1, CHUNK)

  degp = _deg_call(col)
  hw1, dis_row = _tc1_call(x, W1, degp)
  normr = _norm_call(row, col, ea, dis_row.reshape(NP))

  s1 = _spmm_call(hw1, row, col, normr)
  hw2 = _tc_layer_call(s1, b1.reshape(1, H), W2)
  s2 = _spmm_call(hw2, row, col, normr)
  hw3 = _tc_layer_call(s2, b2.reshape(1, H), W3)
  s3 = _spmm_call(hw3, row, col, normr)
  return _tc_final_call(s3, b3.reshape(1, H), batch.reshape(1, N),
                        Wc, bc.reshape(1, C))


# speedup vs baseline: 1.0150x; 1.0150x over previous
import functools

import jax
import jax.numpy as jnp
from jax import lax
from jax.experimental import pallas as pl
from jax.experimental.pallas import tpu as pltpu
from jax.experimental.pallas import tpu_sc as plsc

N = 10000
E = 320000
D_IN = 128
H = 64
C = 100
B = 16

NC = 2
NS = 16
NP = 10240
NPT = NP // NS
EN = E + N
CHUNK = 128
TCH = 81
NW = NC * NS
NROWS = NW * TCH
EP = NROWS * CHUNK
T0 = 94
T1 = 68
TMAX = 94

_SC_PARAMS = pltpu.CompilerParams(needs_layout_passes=False,
                                 use_tc_tiling_on_sc=False)

_mesh = functools.partial(
    plsc.VectorSubcoreMesh, core_axis_name="c", subcore_axis_name="s")

_f32 = jnp.float32
_i32 = jnp.int32


def _wid_base(scale):
  c = lax.axis_index("c")
  s = lax.axis_index("s")
  return (c * NS + s) * scale, c, s


def _prep_body(rowr, colr, ear, normr,
               row_v, col_v, ea_v, dis_v, nbuf, val_v, zero_v, acc):
  base, c, s = _wid_base(TCH)
  w = c * NS + s

  def zstore(i, _):
    zero_v[pl.ds(i * 16, 16)] = jnp.zeros((16,), _f32)
    return 0
  lax.fori_loop(0, NPT // 16, zstore, 0)
  pltpu.sync_copy(zero_v, acc.at[pl.ds(s * NPT, NPT)])
  plsc.subcore_barrier()

  def deg_pass(rowbase):
    pltpu.sync_copy(colr.at[pl.ds(rowbase * TCH, TCH)], col_v)

    def chunk(j, _):
      gbase = (rowbase + j) * CHUNK

      def fill(k, _):
        idx = gbase * 1 + k * 16 + lax.iota(_i32, 16)
        val_v[pl.ds(k * 16, 16)] = jnp.where(idx < EN, jnp.ones((16,), _f32),
                                             jnp.zeros((16,), _f32))
        return 0
      lax.fori_loop(0, CHUNK // 16, fill, 0)
      pltpu.sync_copy(val_v, acc.at[col_v.at[j, 0]], add=True)
      return 0
    lax.fori_loop(0, TCH, chunk, 0)

  deg_pass(w)
  deg_pass((1 - c) * NS + s)
  plsc.subcore_barrier()

  pltpu.sync_copy(acc, dis_v)

  def newton(i, _):
    d = dis_v[pl.ds(i * 16, 16)]
    y = plsc.bitcast(
        jnp.int32(0x5F3759DF) - (plsc.bitcast(d, _i32) >> 1), _f32)
    for _it in range(3):
      y = y * (1.5 - 0.5 * d * y * y)
    dis_v[pl.ds(i * 16, 16)] = y
    return 0
  lax.fori_loop(0, NP // 16, newton, 0)

  pltpu.sync_copy(rowr.at[pl.ds(w * TCH, TCH)], row_v)
  pltpu.sync_copy(colr.at[pl.ds(w * TCH, TCH)], col_v)
  pltpu.sync_copy(ear.at[pl.ds(w * TCH, TCH)], ea_v)

  def chunk(j, _):
    def sub(k, _):
      r16 = row_v[j, 0, pl.ds(k * 16, 16)]
      c16 = col_v[j, 0, pl.ds(k * 16, 16)]
      e16 = ea_v[j, 0, pl.ds(k * 16, 16)]
      dr = plsc.load_gather(dis_v, [r16])
      dc = plsc.load_gather(dis_v, [c16])
      g = (base + j) * CHUNK + k * 16 + lax.iota(_i32, 16)
      nr = dr * dc * jnp.exp(-e16)
      nbuf[j, 0, pl.ds(k * 16, 16)] = jnp.where(g < EN, nr,
                                                jnp.zeros((16,), _f32))
      return 0
    lax.fori_loop(0, CHUNK // 16, sub, 0)
    return 0
  lax.fori_loop(0, TCH, chunk, 0)
  pltpu.sync_copy(nbuf, normr.at[pl.ds(w * TCH, TCH)])


_prep_call = pl.kernel(
    _prep_body,
    out_type=jax.ShapeDtypeStruct((NROWS, 1, CHUNK), _f32),
    mesh=_mesh(),
    compiler_params=_SC_PARAMS,
    scratch_types=[
        pltpu.VMEM((TCH, 1, CHUNK), _i32),
        pltpu.VMEM((TCH, 1, CHUNK), _i32),
        pltpu.VMEM((TCH, 1, CHUNK), _f32),
        pltpu.VMEM((NP,), _f32),
        pltpu.VMEM((TCH, 1, CHUNK), _f32),
        pltpu.VMEM((CHUNK,), _f32),
        pltpu.VMEM((NPT,), _f32),
        pltpu.VMEM_SHARED((NP,), _f32),
    ],
)


NGB = 3
NSB = 3
PD = 2


def _spmm_body(hw, rowr, colr, normr, s_out,
               row_v, col_v, norm_v, gb, sb, zb, acc, gsem, ssem):
  base, c, s = _wid_base(TCH)
  myt = jnp.where(c == 0, T0, T1)
  start = jnp.where(c == 0, s * T0, 16 * T0 + s * T1)

  def zfill(i, _):
    for f in range(4):
      zb[i, pl.ds(f * 16, 16)] = jnp.zeros((16,), _f32)
    return 0
  lax.fori_loop(0, 16, zfill, 0)

  def zfire(t, _):
    pltpu.async_copy(zb, acc.at[pl.ds(s * NPT + t * 16, 16), :], gsem)
    return 0
  lax.fori_loop(0, NPT // 16, zfire, 0)

  def zdrain(t, _):
    pltpu.make_async_copy(zb, acc.at[pl.ds(s * NPT + t * 16, 16), :],
                          gsem).wait()
    return 0
  lax.fori_loop(0, NPT // 16, zdrain, 0)

  pltpu.sync_copy(rowr.at[pl.ds(start, TMAX)], row_v)
  pltpu.sync_copy(colr.at[pl.ds(start, TMAX)], col_v)
  pltpu.sync_copy(normr.at[pl.ds(start, TMAX)], norm_v)
  plsc.subcore_barrier()

  def fire_g(j, bf):
    pltpu.async_copy(hw.at[row_v.at[j, 0]], gb.at[bf], gsem)

  def drain_g(j, bf):
    pltpu.make_async_copy(hw.at[row_v.at[j, 0]], gb.at[bf], gsem).wait()

  def fire_s(j, bf):
    pltpu.async_copy(sb.at[bf], acc.at[col_v.at[j, 0]], ssem, add=True)

  def drain_s(j, bf):
    pltpu.make_async_copy(sb.at[bf], acc.at[col_v.at[j, 0]], ssem).wait()

  for j in range(PD):
    fire_g(j, j)

  def body(j, _):
    gf = j % NGB
    sf = j % NSB

    @pl.when(j >= NSB)
    def _():
      drain_s(j - NSB, sf)

    @pl.when(j + PD < myt)
    def _():
      fire_g(j + PD, (j + PD) % NGB)
    drain_g(j, gf)

    @plsc.parallel_loop(0, CHUNK // 16, unroll=2)
    def _(k):
      nv = norm_v[j, 0, pl.ds(k * 16, 16)]
      for l in range(16):
        nb = nv[jnp.full((16,), l, _i32)]
        r = k * 16 + l
        for f in range(4):
          sb[sf, r, pl.ds(f * 16, 16)] = nb * gb[gf, r, pl.ds(f * 16, 16)]
    fire_s(j, sf)
    return 0
  lax.fori_loop(0, myt, body, 0)
  for jj in range(NSB):
    j = myt - NSB + jj
    drain_s(j, j % NSB)

  plsc.subcore_barrier()
  pltpu.sync_copy(acc.at[pl.ds(s * NPT, NPT), :],
                  s_out.at[c, pl.ds(s * NPT, NPT), :])


_spmm_call = pl.kernel(
    _spmm_body,
    out_type=jax.ShapeDtypeStruct((NC, NP, H), _f32),
    mesh=_mesh(),
    compiler_params=_SC_PARAMS,
    scratch_types=[
        pltpu.VMEM((TMAX, 1, CHUNK), _i32),
        pltpu.VMEM((TMAX, 1, CHUNK), _i32),
        pltpu.VMEM((TMAX, 1, CHUNK), _f32),
        pltpu.VMEM((NGB, CHUNK, H), _f32),
        pltpu.VMEM((NSB, CHUNK, H), _f32),
        pltpu.VMEM((16, H), _f32),
        pltpu.VMEM_SHARED((NP, H), _f32),
        pltpu.SemaphoreType.DMA,
        pltpu.SemaphoreType.DMA,
    ],
)


_DN = (((1,), (1,)), ((), ()))


def _tc1_body(x_ref, w1_ref, hw_ref):
  hw_ref[...] = lax.dot_general(x_ref[...], w1_ref[...], _DN,
                                preferred_element_type=_f32)


def _tc_layer_body(s_ref, b_ref, w_ref, out_ref):
  h = s_ref[0, :N, :] + s_ref[1, :N, :] + b_ref[...]
  h = jnp.maximum(h, 0.0)
  out_ref[...] = lax.dot_general(h, w_ref[...], _DN,
                                 preferred_element_type=_f32)


def _tc_final_body(s_ref, b_ref, batch_ref, wc_ref, bc_ref, out_ref):
  h = s_ref[0, :N, :] + s_ref[1, :N, :] + b_ref[...]
  h = jnp.maximum(h, 0.0)
  bt = batch_ref[...]
  oht = (bt == lax.broadcasted_iota(_i32, (B, N), 0)).astype(_f32)
  sums = lax.dot_general(oht, h, (((1,), (0,)), ((), ())),
                         preferred_element_type=_f32)
  cnt = jnp.sum(oht, axis=1, keepdims=True)
  pooled = sums / jnp.maximum(cnt, 1.0)
  out_ref[...] = lax.dot_general(pooled, wc_ref[...], _DN,
                                 preferred_element_type=_f32) + bc_ref[...]


_tc1_call = pl.pallas_call(
    _tc1_body,
    out_shape=jax.ShapeDtypeStruct((N, H), _f32),
)

_tc_layer_call = pl.pallas_call(
    _tc_layer_body,
    out_shape=jax.ShapeDtypeStruct((N, H), _f32),
)

_tc_final_call = pl.pallas_call(
    _tc_final_body,
    out_shape=jax.ShapeDtypeStruct((B, C), _f32),
)


def kernel(x, edge_index, edge_attr, batch, W1, b1, W2, b2, W3, b3, Wc, bc):
  loop = jnp.arange(N, dtype=_i32)
  zpad_i = jnp.zeros((EP - EN,), _i32)
  row = jnp.concatenate([edge_index[0], loop, zpad_i]).reshape(NROWS, 1, CHUNK)
  col = jnp.concatenate([edge_index[1], loop, zpad_i]).reshape(NROWS, 1, CHUNK)
  ea = jnp.concatenate([edge_attr,
                        jnp.zeros((EP - E,), _f32)]).reshape(NROWS, 1, CHUNK)

  normr = _prep_call(row, col, ea)
  hw1 = _tc1_call(x, W1)

  s1 = _spmm_call(hw1, row, col, normr)
  hw2 = _tc_layer_call(s1, b1.reshape(1, H), W2)
  s2 = _spmm_call(hw2, row, col, normr)
  hw3 = _tc_layer_call(s2, b2.reshape(1, H), W3)
  s3 = _spmm_call(hw3, row, col, normr)
  return _tc_final_call(s3, b3.reshape(1, H), batch.reshape(1, N),
                        Wc, bc.reshape(1, C))
